# baseline (speedup 1.0000x reference)
import functools

import jax
import jax.numpy as jnp
from jax.experimental import pallas as pl
from jax.experimental.pallas import tpu as pltpu

_VMEM_LIMIT = 64 * 1024 * 1024


def _largest_divisor_leq(n, k):
    k = max(1, min(n, k))
    while n % k:
        k -= 1
    return k


def _se_kernel(x_ref, w1t_ref, w2_ref, o_ref, *, inv_hw):
    x = x_ref[...]
    y = jnp.sum(x, axis=-1, keepdims=True) * inv_hw
    z1 = jnp.maximum(jnp.sum(w1t_ref[...] * y, axis=1, keepdims=True), 0.0)
    z2 = jnp.sum(w2_ref[...] * z1, axis=-1, keepdims=True)
    o_ref[...] = x * jax.nn.sigmoid(z2)


def kernel(x_nchw, fc1_w_t, fc2_w):
    B, C, H, W = x_nchw.shape
    C1, Cr = fc1_w_t.shape
    assert C1 == C and fc2_w.shape == (C, Cr)
    HW = H * W
    x = x_nchw.reshape(B, C, HW)

    itemsize = x_nchw.dtype.itemsize
    per_batch_bytes = C * HW * itemsize
    tb = _largest_divisor_leq(B, max(1, (8 << 20) // per_batch_bytes))

    out = pl.pallas_call(
        functools.partial(_se_kernel, inv_hw=1.0 / HW),
        out_shape=jax.ShapeDtypeStruct((B, C, HW), x.dtype),
        grid=(B // tb,),
        in_specs=[
            pl.BlockSpec((tb, C, HW), lambda b: (b, 0, 0)),
            pl.BlockSpec((C, Cr), lambda b: (0, 0)),
            pl.BlockSpec((C, Cr), lambda b: (0, 0)),
        ],
        out_specs=pl.BlockSpec((tb, C, HW), lambda b: (b, 0, 0)),
        compiler_params=pltpu.CompilerParams(
            dimension_semantics=("parallel",),
            vmem_limit_bytes=_VMEM_LIMIT),
    )(x, fc1_w_t, fc2_w)
    return out.reshape(B, C, H, W)
